# Initial kernel scaffold; baseline (speedup 1.0000x reference)
import jax
import jax.numpy as jnp
from jax.experimental import pallas as pl


def kernel(x, edge_index, W1, b1, W2, b2):
    raise NotImplementedError("write your pallas kernel here")


# speedup vs baseline: 11.7868x; 11.7868x over previous
import functools

import jax
import jax.numpy as jnp
from jax import lax
from jax.experimental import pallas as pl
from jax.experimental.pallas import tpu as pltpu
from jax.experimental.pallas import tpu_sc as plsc

N = 10000
D = 128
E = 320000

NC = 2
NS = 16
NW = NC * NS

C = 128
K = 79
EPAD = NW * K * C - E
ZR = K * C
RS = ZR // NS
DEGW = 128

_MESH = plsc.VectorSubcoreMesh(
    core_axis_name="c", subcore_axis_name="s", num_cores=NC, num_subcores=NS
)


def _worker_id():
    return lax.axis_index("c") * NS + lax.axis_index("s")


def _sc_deg_body(dst_hbm, ones_hbm, zeros_hbm, deg_out, dst_v, ones_v, shared_deg):
    c = lax.axis_index("c")
    s = lax.axis_index("s")
    wid = _worker_id()
    pltpu.sync_copy(zeros_hbm.at[pl.ds(s * RS, RS)], shared_deg.at[pl.ds(s * RS, RS)])
    pltpu.sync_copy(dst_hbm.at[wid], dst_v)
    pltpu.sync_copy(ones_hbm, ones_v)
    plsc.subcore_barrier()

    def step(j, carry):
        pltpu.sync_copy(ones_v, shared_deg.at[dst_v.at[j]], add=True)
        return carry

    lax.fori_loop(0, K, step, 0)
    plsc.subcore_barrier()
    pltpu.sync_copy(shared_deg.at[pl.ds(s * RS, RS)], deg_out.at[c, pl.ds(s * RS, RS)])


@jax.jit
def _sc_deg(dst3, ones16, zeros16):
    return pl.kernel(
        _sc_deg_body,
        out_type=jax.ShapeDtypeStruct((NC, ZR, DEGW), jnp.float32),
        mesh=_MESH,
        scratch_types=[
            pltpu.VMEM((K, C), jnp.int32),
            pltpu.VMEM((C, DEGW), jnp.float32),
            pltpu.VMEM_SHARED((ZR, DEGW), jnp.float32),
        ],
    )(dst3, ones16, zeros16)


def _sc_edge_body(y_hbm, src_hbm, dst_hbm, zeros_hbm, z_out,
                  src_v, dst_v, rows_v, shared_z, sem):
    c = lax.axis_index("c")
    s = lax.axis_index("s")
    wid = _worker_id()
    pltpu.sync_copy(zeros_hbm.at[pl.ds(s * RS, RS)], shared_z.at[pl.ds(s * RS, RS)])
    pltpu.sync_copy(src_hbm.at[wid], src_v)
    pltpu.sync_copy(dst_hbm.at[wid], dst_v)
    plsc.subcore_barrier()

    def step(j, carry):
        pltpu.async_copy(y_hbm.at[src_v.at[j]], rows_v, sem).wait()
        pltpu.sync_copy(rows_v, shared_z.at[dst_v.at[j]], add=True)
        return carry

    lax.fori_loop(0, K, step, 0)
    plsc.subcore_barrier()
    pltpu.sync_copy(shared_z.at[pl.ds(s * RS, RS)], z_out.at[c, pl.ds(s * RS, RS)])


@jax.jit
def _sc_edge(y, src3, dst3, zerosD):
    return pl.kernel(
        _sc_edge_body,
        out_type=jax.ShapeDtypeStruct((NC, ZR, D), jnp.float32),
        mesh=_MESH,
        scratch_types=[
            pltpu.VMEM((K, C), jnp.int32),
            pltpu.VMEM((K, C), jnp.int32),
            pltpu.VMEM((C, D), jnp.float32),
            pltpu.VMEM_SHARED((ZR, D), jnp.float32),
            pltpu.SemaphoreType.DMA,
        ],
    )(y, src3, dst3, zerosD)


def _tc_mm_body(x_ref, w_ref, o_ref):
    o_ref[...] = jnp.dot(x_ref[...], w_ref[...], preferred_element_type=jnp.float32)


@jax.jit
def _tc_mm(x, w):
    return pl.pallas_call(
        _tc_mm_body,
        out_shape=jax.ShapeDtypeStruct((N, D), jnp.float32),
    )(x, w)


def _tc_scale_body(degp_ref, xw_ref, y_ref, dinv_ref):
    deg = degp_ref[0, :, 0:1] + degp_ref[1, :, 0:1]
    dinv = lax.rsqrt(deg + 1.0)
    dv = dinv[:N, :]
    y_ref[...] = xw_ref[...] * dv
    dinv_ref[...] = dv


@jax.jit
def _tc_scale(deg_parts, xw):
    return pl.pallas_call(
        _tc_scale_body,
        out_shape=(
            jax.ShapeDtypeStruct((N, D), jnp.float32),
            jax.ShapeDtypeStruct((N, 1), jnp.float32),
        ),
    )(deg_parts, xw)


def _tc_comb_mm_body(z_ref, y_ref, dinv_ref, b_ref, w_ref, o_ref):
    z = z_ref[0, :N, :] + z_ref[1, :N, :]
    h = (z + y_ref[...]) * dinv_ref[...] + b_ref[...]
    h = jnp.maximum(h, 0.0)
    o_ref[...] = (
        jnp.dot(h, w_ref[...], preferred_element_type=jnp.float32) * dinv_ref[...]
    )


@jax.jit
def _tc_comb_mm(z, y, dinv, b, w):
    return pl.pallas_call(
        _tc_comb_mm_body,
        out_shape=jax.ShapeDtypeStruct((N, D), jnp.float32),
    )(z, y, dinv, b, w)


def _tc_comb_body(z_ref, y_ref, dinv_ref, b_ref, o_ref):
    z = z_ref[0, :N, :] + z_ref[1, :N, :]
    o_ref[...] = (z + y_ref[...]) * dinv_ref[...] + b_ref[...]


@jax.jit
def _tc_comb(z, y, dinv, b):
    return pl.pallas_call(
        _tc_comb_body,
        out_shape=jax.ShapeDtypeStruct((N, D), jnp.float32),
    )(z, y, dinv, b)


def kernel(x, edge_index, W1, b1, W2, b2):
    src = edge_index[0].astype(jnp.int32)
    dst = edge_index[1].astype(jnp.int32)
    src3 = jnp.concatenate([src, jnp.zeros((EPAD,), jnp.int32)]).reshape(NW, K, C)
    dst3 = jnp.concatenate([dst, jnp.full((EPAD,), N, jnp.int32)]).reshape(NW, K, C)
    onesD = jnp.ones((C, DEGW), jnp.float32)
    zerosD = jnp.zeros((ZR, D), jnp.float32)

    deg_parts = _sc_deg(dst3, onesD, zerosD)
    xw1 = _tc_mm(x, W1)
    y1, dinv = _tc_scale(deg_parts, xw1)
    z1 = _sc_edge(y1, src3, dst3, zerosD)
    y2 = _tc_comb_mm(z1, y1, dinv, b1.reshape(1, D), W2)
    z2 = _sc_edge(y2, src3, dst3, zerosD)
    return _tc_comb(z2, y2, dinv, b2.reshape(1, D))
